# baseline (speedup 1.0000x reference)
import jax
import jax.numpy as jnp
from jax import lax
from jax.experimental import pallas as pl
from jax.experimental.pallas import tpu as pltpu


def _caps_body(nb, K, NT, Ho, WT, Cin2, PAD):
    RB = Ho * WT

    def body(xp_ref, w_ref, b_ref, seg_ref, o_ref):
        pieces = []
        for i in range(K):
            pi, r0 = i % 2, i // 2
            for m in range(NT):
                s = xp_ref[:, pi, r0:r0 + Ho, m, :, :]
                pieces.append(s.reshape(nb * RB, Cin2))
        patches = jnp.concatenate(pieces, axis=1)
        acc = jnp.dot(patches, w_ref[...],
                      preferred_element_type=jnp.float32)
        acc = acc + b_ref[...]
        sq = jnp.dot((acc * acc).astype(jnp.bfloat16), seg_ref[...],
                     preferred_element_type=jnp.float32)
        o = acc * (jnp.sqrt(sq) / (1.0 + sq))
        o_ref[...] = o.reshape(nb, RB, PAD).astype(jnp.bfloat16)

    return body


def _primary_caps(x, weight, bias, OC=8, K=9, S=2):
    B, Cin, H, W = x.shape
    KK = K * K
    Ho = (H - K) // S + 1
    Wo = (W - K) // S + 1
    HoWo = Ho * Wo
    Hh, Wh = H // 2, W // 2
    NT = K // 2 + 1
    WT = 8
    Wp = NT - 1 + WT
    Cin2 = 2 * Cin
    CKKP = K * NT * Cin2
    PAD = max(128, ((KK * OC + 127) // 128) * 128)

    xr = jnp.transpose(x, (0, 2, 3, 1)).reshape(B, Hh, 2, Wh, 2, Cin)
    xp = jnp.transpose(xr, (0, 2, 1, 3, 4, 5)).reshape(B, 2, Hh, Wh, Cin2)
    xp = jnp.pad(xp, ((0, 0), (0, 0), (0, 0), (0, Wp - Wh), (0, 0)))
    xp = xp.astype(jnp.bfloat16)
    xp = jnp.stack([xp[:, :, :, m:m + WT, :] for m in range(NT)], axis=3)

    wr = jnp.transpose(weight, (2, 3, 1, 0))
    wr = wr.reshape(K, K, Cin, OC, KK)
    wr = jnp.transpose(wr, (0, 1, 2, 4, 3)).reshape(K, K, Cin, KK * OC)
    wr = jnp.pad(wr, ((0, 0), (0, 2 * NT - K), (0, 0), (0, 0)))
    wr = wr.reshape(CKKP, KK * OC)
    wr = jnp.pad(wr, ((0, 0), (0, PAD - KK * OC))).astype(jnp.bfloat16)

    b = bias.reshape(OC, KK).transpose(1, 0).reshape(KK * OC)
    b = jnp.pad(b, (0, PAD - KK * OC)).astype(jnp.float32).reshape(1, PAD)

    rg = lax.broadcasted_iota(jnp.int32, (PAD, PAD), 0) // OC
    cg = lax.broadcasted_iota(jnp.int32, (PAD, PAD), 1) // OC
    seg = (rg == cg).astype(jnp.bfloat16)

    nb = next(g for g in (16, 8, 4, 2, 1) if B % g == 0)
    RB = Ho * WT
    out = pl.pallas_call(
        _caps_body(nb, K, NT, Ho, WT, Cin2, PAD),
        out_shape=jax.ShapeDtypeStruct((B, RB, PAD), jnp.bfloat16),
        grid=(B // nb,),
        in_specs=[
            pl.BlockSpec((nb, 2, Hh, NT, WT, Cin2),
                         lambda g: (g, 0, 0, 0, 0, 0)),
            pl.BlockSpec((CKKP, PAD), lambda g: (0, 0)),
            pl.BlockSpec((1, PAD), lambda g: (0, 0)),
            pl.BlockSpec((PAD, PAD), lambda g: (0, 0)),
        ],
        out_specs=pl.BlockSpec((nb, RB, PAD), lambda g: (g, 0, 0)),
        compiler_params=pltpu.CompilerParams(
            dimension_semantics=("parallel",),
            vmem_limit_bytes=56 << 20),
    )(xp, wr, b, seg)

    out = out.reshape(B, Ho, WT, PAD)[:, :, :Wo, :KK * OC]
    out = out.reshape(B, HoWo, KK, OC)
    out = jnp.transpose(out, (0, 2, 1, 3)).reshape(B, KK * HoWo, OC)
    return out.astype(jnp.float32)


def kernel(x, weight, bias):
    return _primary_caps(x, weight, bias)


# speedup vs baseline: 1.2458x; 1.2458x over previous
import jax
import jax.numpy as jnp
from jax import lax
from jax.experimental import pallas as pl
from jax.experimental.pallas import tpu as pltpu


def _caps_body(nb, K, NT, Ho, WT, Cin2, PAD):
    RB = Ho * WT

    def body(xp_ref, w_ref, b_ref, seg_ref, o_ref):
        split = (K + 1) // 2
        acc = b_ref[...]
        for lo, hi in ((0, split), (split, K)):
            pieces = []
            for i in range(lo, hi):
                pi, r0 = i % 2, i // 2
                for m in range(NT):
                    s = xp_ref[:, pi, r0:r0 + Ho, m:m + WT, :]
                    pieces.append(s.reshape(nb * RB, Cin2))
            chunk = jnp.concatenate(pieces, axis=1)
            kcols = NT * Cin2
            acc = acc + jnp.dot(chunk, w_ref[lo * kcols:hi * kcols, :],
                                preferred_element_type=jnp.float32)
        sq = jnp.dot((acc * acc).astype(jnp.bfloat16), seg_ref[...],
                     preferred_element_type=jnp.float32)
        o = acc * (jnp.sqrt(sq) * pl.reciprocal(1.0 + sq, approx=True))
        o_ref[...] = o.reshape(nb, RB, PAD)

    return body


def _primary_caps(x, weight, bias, OC=8, K=9, S=2):
    B, Cin, H, W = x.shape
    KK = K * K
    Ho = (H - K) // S + 1
    Wo = (W - K) // S + 1
    HoWo = Ho * Wo
    Hh, Wh = H // 2, W // 2
    NT = K // 2 + 1
    WT = 8
    Wp = NT - 1 + WT
    Cin2 = 2 * Cin
    CKKP = K * NT * Cin2
    PAD = max(128, ((KK * OC + 127) // 128) * 128)

    xr = jnp.transpose(x, (0, 2, 3, 1)).reshape(B, Hh, 2, Wh, 2, Cin)
    xp = jnp.transpose(xr, (0, 2, 1, 3, 4, 5)).reshape(B, 2, Hh, Wh, Cin2)
    xp = jnp.pad(xp, ((0, 0), (0, 0), (0, 0), (0, Wp - Wh), (0, 0)))
    xp = xp.astype(jnp.bfloat16)

    wr = jnp.transpose(weight, (2, 3, 1, 0))
    wr = wr.reshape(K, K, Cin, OC, KK)
    wr = jnp.transpose(wr, (0, 1, 2, 4, 3)).reshape(K, K, Cin, KK * OC)
    wr = jnp.pad(wr, ((0, 0), (0, 2 * NT - K), (0, 0), (0, 0)))
    wr = wr.reshape(CKKP, KK * OC)
    wr = jnp.pad(wr, ((0, 0), (0, PAD - KK * OC))).astype(jnp.bfloat16)

    b = bias.reshape(OC, KK).transpose(1, 0).reshape(KK * OC)
    b = jnp.pad(b, (0, PAD - KK * OC)).astype(jnp.float32).reshape(1, PAD)

    rg = lax.broadcasted_iota(jnp.int32, (PAD, PAD), 0) // OC
    cg = lax.broadcasted_iota(jnp.int32, (PAD, PAD), 1) // OC
    seg = (rg == cg).astype(jnp.bfloat16)

    nb = next(g for g in (16, 8, 4, 2, 1) if B % g == 0)
    RB = Ho * WT
    out = pl.pallas_call(
        _caps_body(nb, K, NT, Ho, WT, Cin2, PAD),
        out_shape=jax.ShapeDtypeStruct((B, RB, PAD), jnp.float32),
        grid=(B // nb,),
        in_specs=[
            pl.BlockSpec((nb, 2, Hh, Wp, Cin2), lambda g: (g, 0, 0, 0, 0)),
            pl.BlockSpec((CKKP, PAD), lambda g: (0, 0)),
            pl.BlockSpec((1, PAD), lambda g: (0, 0)),
            pl.BlockSpec((PAD, PAD), lambda g: (0, 0)),
        ],
        out_specs=pl.BlockSpec((nb, RB, PAD), lambda g: (g, 0, 0)),
        compiler_params=pltpu.CompilerParams(
            dimension_semantics=("parallel",),
            vmem_limit_bytes=56 << 20),
    )(xp, wr, b, seg)

    out = out.reshape(B, Ho, WT, PAD)[:, :, :Wo, :KK * OC]
    out = out.reshape(B, HoWo, KK, OC)
    out = jnp.transpose(out, (0, 2, 1, 3)).reshape(B, KK * HoWo, OC)
    return out


def kernel(x, weight, bias):
    return _primary_caps(x, weight, bias)


# speedup vs baseline: 1.2732x; 1.0220x over previous
import jax
import jax.numpy as jnp
from jax import lax
from jax.experimental import pallas as pl
from jax.experimental.pallas import tpu as pltpu


def _caps_body(nb, K, NT, Ho, WT, Cin2, PAD):
    RB = Ho * WT

    def body(xp_ref, w_ref, b_ref, seg_ref, o_ref):
        acc = b_ref[...]
        for lo, hi in ((0, 3), (3, 6), (6, K)):
            pieces = []
            for i in range(lo, hi):
                pi, r0 = i % 2, i // 2
                for m in range(NT):
                    s = xp_ref[:, pi, r0:r0 + Ho, m:m + WT, :]
                    pieces.append(s.reshape(nb * RB, Cin2))
            chunk = jnp.concatenate(pieces, axis=1)
            kcols = NT * Cin2
            acc = acc + jnp.dot(chunk, w_ref[lo * kcols:hi * kcols, :],
                                preferred_element_type=jnp.float32)
        sq = jnp.dot((acc * acc).astype(jnp.bfloat16), seg_ref[...],
                     preferred_element_type=jnp.float32)
        o = acc * (jnp.sqrt(sq) * pl.reciprocal(1.0 + sq, approx=True))
        o_ref[...] = o.reshape(nb, RB, PAD)

    return body


def _primary_caps(x, weight, bias, OC=8, K=9, S=2):
    B, Cin, H, W = x.shape
    KK = K * K
    Ho = (H - K) // S + 1
    Wo = (W - K) // S + 1
    HoWo = Ho * Wo
    Hh, Wh = H // 2, W // 2
    NT = K // 2 + 1
    WT = 8
    Wp = NT - 1 + WT
    Cin2 = 2 * Cin
    CKKP = K * NT * Cin2
    PAD = max(128, ((KK * OC + 127) // 128) * 128)

    xr = jnp.transpose(x, (0, 2, 3, 1)).reshape(B, Hh, 2, Wh, 2, Cin)
    xp = jnp.transpose(xr, (0, 2, 1, 3, 4, 5)).reshape(B, 2, Hh, Wh, Cin2)
    xp = jnp.pad(xp, ((0, 0), (0, 0), (0, 0), (0, Wp - Wh), (0, 0)))
    xp = xp.astype(jnp.bfloat16)

    wr = jnp.transpose(weight, (2, 3, 1, 0))
    wr = wr.reshape(K, K, Cin, OC, KK)
    wr = jnp.transpose(wr, (0, 1, 2, 4, 3)).reshape(K, K, Cin, KK * OC)
    wr = jnp.pad(wr, ((0, 0), (0, 2 * NT - K), (0, 0), (0, 0)))
    wr = wr.reshape(CKKP, KK * OC)
    wr = jnp.pad(wr, ((0, 0), (0, PAD - KK * OC))).astype(jnp.bfloat16)

    b = bias.reshape(OC, KK).transpose(1, 0).reshape(KK * OC)
    b = jnp.pad(b, (0, PAD - KK * OC)).astype(jnp.float32).reshape(1, PAD)

    rg = lax.broadcasted_iota(jnp.int32, (PAD, PAD), 0) // OC
    cg = lax.broadcasted_iota(jnp.int32, (PAD, PAD), 1) // OC
    seg = (rg == cg).astype(jnp.bfloat16)

    nb = next(g for g in (16, 8, 4, 2, 1) if B % g == 0)
    RB = Ho * WT
    out = pl.pallas_call(
        _caps_body(nb, K, NT, Ho, WT, Cin2, PAD),
        out_shape=jax.ShapeDtypeStruct((B, RB, PAD), jnp.float32),
        grid=(B // nb,),
        in_specs=[
            pl.BlockSpec((nb, 2, Hh, Wp, Cin2), lambda g: (g, 0, 0, 0, 0)),
            pl.BlockSpec((CKKP, PAD), lambda g: (0, 0)),
            pl.BlockSpec((1, PAD), lambda g: (0, 0)),
            pl.BlockSpec((PAD, PAD), lambda g: (0, 0)),
        ],
        out_specs=pl.BlockSpec((nb, RB, PAD), lambda g: (g, 0, 0)),
        compiler_params=pltpu.CompilerParams(
            dimension_semantics=("parallel",),
            vmem_limit_bytes=56 << 20),
    )(xp, wr, b, seg)

    out = out.reshape(B, Ho, WT, PAD)[:, :, :Wo, :KK * OC]
    out = out.reshape(B, HoWo, KK, OC)
    out = jnp.transpose(out, (0, 2, 1, 3)).reshape(B, KK * HoWo, OC)
    return out


def kernel(x, weight, bias):
    return _primary_caps(x, weight, bias)
